# baseline (speedup 1.0000x reference)
import functools

import jax
import jax.numpy as jnp
from jax import lax
from jax.experimental import pallas as pl
from jax.experimental.pallas import tpu as pltpu
from jax.experimental.pallas import tpu_sc as plsc

POOL = 256
K = 8
PLEN = 8
DIM = 768
BATCH = 128
ROW = PLEN * DIM

NC = 2
NS = 16
NW = NC * NS
ROWS_PER_W = (BATCH * K) // NW
CHUNK = 8


def _topk_body(q_ref, k_ref, sim_ref, idx_ref):
    q = q_ref[...]
    k = k_ref[...]
    eps = 1e-8
    qn = jnp.maximum(jnp.sqrt(jnp.sum(q * q, axis=1, keepdims=True)), eps)
    ones = jnp.ones((1, DIM), jnp.float32)
    kn2 = lax.dot_general(ones, k * k, (((1,), (1,)), ((), ())),
                          preferred_element_type=jnp.float32)
    kn = jnp.maximum(jnp.sqrt(kn2), eps)
    dot = lax.dot_general(q, k, (((1,), (1,)), ((), ())),
                          preferred_element_type=jnp.float32)
    m = 1.0 - dot / (qn * kn)

    iota = lax.broadcasted_iota(jnp.int32, (BATCH, POOL), 1)
    col = lax.broadcasted_iota(jnp.int32, (BATCH, K), 1)
    simacc = jnp.zeros((BATCH, K), jnp.float32)
    idxacc = jnp.zeros((BATCH, K), jnp.int32)
    for j in range(K):
        mn = jnp.min(m, axis=1, keepdims=True)
        sel = jnp.min(jnp.where(m == mn, iota, POOL), axis=1,
                      keepdims=True)
        simacc = jnp.where(col == j, mn, simacc)
        idxacc = jnp.where(col == j, sel, idxacc)
        m = jnp.where(iota == sel, jnp.inf, m)
    sim_ref[...] = simacc
    idx_ref[...] = idxacc


def _gather_body(prompts_hbm, idx_hbm, out_hbm, idx_v, buf, sem):
    wid = lax.axis_index("s") * NC + lax.axis_index("c")
    base = wid * ROWS_PER_W
    pltpu.sync_copy(idx_hbm.at[pl.ds(base, ROWS_PER_W)], idx_v)
    for c in range(ROWS_PER_W // CHUNK):
        pltpu.async_copy(
            prompts_hbm.at[idx_v.at[pl.ds(c * CHUNK, CHUNK)]], buf, sem
        ).wait()
        pltpu.sync_copy(buf, out_hbm.at[pl.ds(base + c * CHUNK, CHUNK)])


@jax.jit
def kernel(query, key_param, prompts):
    sim, idx = pl.pallas_call(
        _topk_body,
        out_shape=(
            jax.ShapeDtypeStruct((BATCH, K), jnp.float32),
            jax.ShapeDtypeStruct((BATCH, K), jnp.int32),
        ),
    )(query, key_param)

    gather = pl.kernel(
        _gather_body,
        out_type=jax.ShapeDtypeStruct((BATCH * K, ROW), jnp.float32),
        mesh=plsc.VectorSubcoreMesh(core_axis_name="c", subcore_axis_name="s"),
        scratch_types=[
            pltpu.VMEM((ROWS_PER_W,), jnp.int32),
            pltpu.VMEM((CHUNK, ROW), jnp.float32),
            pltpu.SemaphoreType.DMA,
        ],
    )
    sel_flat = gather(prompts.reshape(POOL, ROW), idx.reshape(-1))
    return sim, sel_flat.reshape(BATCH, K, PLEN, DIM)


# speedup vs baseline: 1.5880x; 1.5880x over previous
import functools

import jax
import jax.numpy as jnp
from jax import lax
from jax.experimental import pallas as pl
from jax.experimental.pallas import tpu as pltpu
from jax.experimental.pallas import tpu_sc as plsc

POOL = 256
K = 8
PLEN = 8
DIM = 768
BATCH = 128
ROW = PLEN * DIM

NC = 2
NS = 16
NW = NC * NS
ROWS_PER_W = (BATCH * K) // NW
CHUNK = 8


def _topk_body(q_ref, k_ref, qn_ref, kn_ref, sim_ref, idx_ref):
    dot = lax.dot_general(q_ref[...], k_ref[...], (((1,), (1,)), ((), ())),
                          preferred_element_type=jnp.float32)
    m = 1.0 - dot / (qn_ref[...] * kn_ref[...])

    iota = lax.broadcasted_iota(jnp.int32, (BATCH, POOL), 1)
    col = lax.broadcasted_iota(jnp.int32, (BATCH, K), 1)
    simacc = jnp.zeros((BATCH, K), jnp.float32)
    idxacc = jnp.zeros((BATCH, K), jnp.int32)
    for j in range(K):
        mn = jnp.min(m, axis=1, keepdims=True)
        sel = jnp.min(jnp.where(m == mn, iota, POOL), axis=1,
                      keepdims=True)
        simacc = jnp.where(col == j, mn, simacc)
        idxacc = jnp.where(col == j, sel, idxacc)
        m = jnp.where(iota == sel, jnp.inf, m)
    sim_ref[...] = simacc
    idx_ref[...] = idxacc


def _gather_body(prompts_hbm, idx_hbm, out_hbm, idx_v, buf, sem):
    wid = lax.axis_index("s") * NC + lax.axis_index("c")
    base = wid * ROWS_PER_W
    pltpu.sync_copy(idx_hbm.at[pl.ds(base, ROWS_PER_W)], idx_v)
    for c in range(ROWS_PER_W // CHUNK):
        pltpu.async_copy(
            prompts_hbm.at[idx_v.at[pl.ds(c * CHUNK, CHUNK)]], buf, sem
        ).wait()
        pltpu.sync_copy(buf, out_hbm.at[pl.ds(base + c * CHUNK, CHUNK)])


@jax.jit
def kernel(query, key_param, prompts):
    eps = 1e-8
    qn = jnp.maximum(jnp.linalg.norm(query, axis=-1, keepdims=True), eps)
    kn = jnp.maximum(jnp.linalg.norm(key_param, axis=-1, keepdims=True), eps)
    sim, idx = pl.pallas_call(
        _topk_body,
        out_shape=(
            jax.ShapeDtypeStruct((BATCH, K), jnp.float32),
            jax.ShapeDtypeStruct((BATCH, K), jnp.int32),
        ),
    )(query, key_param, qn, kn.T)

    gather = pl.kernel(
        _gather_body,
        out_type=jax.ShapeDtypeStruct((BATCH * K, PLEN, DIM), jnp.float32),
        mesh=plsc.VectorSubcoreMesh(core_axis_name="c", subcore_axis_name="s"),
        scratch_types=[
            pltpu.VMEM((ROWS_PER_W,), jnp.int32),
            pltpu.VMEM((CHUNK, PLEN, DIM), jnp.float32),
            pltpu.SemaphoreType.DMA,
        ],
    )
    sel_flat = gather(prompts, idx.reshape(-1))
    return sim, sel_flat.reshape(BATCH, K, PLEN, DIM)


# speedup vs baseline: 1.6141x; 1.0164x over previous
import functools

import jax
import jax.numpy as jnp
from jax import lax
from jax.experimental import pallas as pl
from jax.experimental.pallas import tpu as pltpu
from jax.experimental.pallas import tpu_sc as plsc

POOL = 256
K = 8
PLEN = 8
DIM = 768
BATCH = 128
ROW = PLEN * DIM

NC = 2
NS = 16
NW = NC * NS
ROWS_PER_W = (BATCH * K) // NW
CHUNK = 8
NBUF = 2
NCH = ROWS_PER_W // CHUNK


def _topk_body(q_ref, k_ref, qn_ref, kn_ref, sim_ref, idx_ref):
    dot = lax.dot_general(q_ref[...], k_ref[...], (((1,), (1,)), ((), ())),
                          preferred_element_type=jnp.float32)
    m = 1.0 - dot / (qn_ref[...] * kn_ref[...])

    iota = lax.broadcasted_iota(jnp.int32, (BATCH, POOL), 1)
    col = lax.broadcasted_iota(jnp.int32, (BATCH, K), 1)
    simacc = jnp.zeros((BATCH, K), jnp.float32)
    idxacc = jnp.zeros((BATCH, K), jnp.int32)
    for j in range(K):
        mn = jnp.min(m, axis=1, keepdims=True)
        sel = jnp.min(jnp.where(m == mn, iota, POOL), axis=1,
                      keepdims=True)
        simacc = jnp.where(col == j, mn, simacc)
        idxacc = jnp.where(col == j, sel, idxacc)
        m = jnp.where(iota == sel, jnp.inf, m)
    sim_ref[...] = simacc
    idx_ref[...] = idxacc


def _gather_body(prompts_hbm, idx_hbm, out_hbm, idx_v, bufs, gsems, ssems):
    wid = lax.axis_index("s") * NC + lax.axis_index("c")
    base = wid * ROWS_PER_W
    pltpu.sync_copy(idx_hbm.at[pl.ds(base, ROWS_PER_W)], idx_v)

    def gather(c):
        return pltpu.async_copy(
            prompts_hbm.at[idx_v.at[pl.ds(c * CHUNK, CHUNK)]],
            bufs[c % NBUF], gsems[c % NBUF])

    def scatter(c):
        return pltpu.async_copy(
            bufs[c % NBUF], out_hbm.at[pl.ds(base + c * CHUNK, CHUNK)],
            ssems[c % NBUF])

    g = [gather(b) for b in range(NBUF)]
    s = [None] * NCH
    for c in range(NCH):
        g[c % NBUF].wait()
        s[c] = scatter(c)
        if c + NBUF < NCH:
            s[c].wait()
            g[c % NBUF] = gather(c + NBUF)
    for c in range(NCH - NBUF, NCH):
        s[c].wait()


@jax.jit
def kernel(query, key_param, prompts):
    eps = 1e-8
    qn = jnp.maximum(jnp.linalg.norm(query, axis=-1, keepdims=True), eps)
    kn = jnp.maximum(jnp.linalg.norm(key_param, axis=-1, keepdims=True), eps)
    sim, idx = pl.pallas_call(
        _topk_body,
        out_shape=(
            jax.ShapeDtypeStruct((BATCH, K), jnp.float32),
            jax.ShapeDtypeStruct((BATCH, K), jnp.int32),
        ),
    )(query, key_param, qn, kn.T)

    gather = pl.kernel(
        _gather_body,
        out_type=jax.ShapeDtypeStruct((BATCH * K, PLEN, DIM), jnp.float32),
        mesh=plsc.VectorSubcoreMesh(core_axis_name="c", subcore_axis_name="s"),
        scratch_types=[
            pltpu.VMEM((ROWS_PER_W,), jnp.int32),
            [pltpu.VMEM((CHUNK, PLEN, DIM), jnp.float32) for _ in range(NBUF)],
            [pltpu.SemaphoreType.DMA for _ in range(NBUF)],
            [pltpu.SemaphoreType.DMA for _ in range(NBUF)],
        ],
    )
    sel_flat = gather(prompts, idx.reshape(-1))
    return sim, sel_flat.reshape(BATCH, K, PLEN, DIM)


# speedup vs baseline: 1.6257x; 1.0072x over previous
import functools

import jax
import jax.numpy as jnp
from jax import lax
from jax.experimental import pallas as pl
from jax.experimental.pallas import tpu as pltpu
from jax.experimental.pallas import tpu_sc as plsc

POOL = 256
K = 8
PLEN = 8
DIM = 768
BATCH = 128
ROW = PLEN * DIM

NC = 2
NS = 16
NW = NC * NS
ROWS_PER_W = (BATCH * K) // NW
CHUNK = 4
NBUF = 4
NCH = ROWS_PER_W // CHUNK
IDX_STRIDE = 8


def _topk_body(q_ref, k_ref, qn_ref, kn_ref, sim_ref, idx_ref):
    dot = lax.dot_general(q_ref[...], k_ref[...], (((1,), (1,)), ((), ())),
                          preferred_element_type=jnp.float32)
    m = 1.0 - dot / (qn_ref[...] * kn_ref[...])

    iota = lax.broadcasted_iota(jnp.int32, (BATCH, POOL), 1)
    col = lax.broadcasted_iota(jnp.int32, (BATCH, K), 1)
    simacc = jnp.zeros((BATCH, K), jnp.float32)
    idxacc = jnp.zeros((BATCH, K), jnp.int32)
    for j in range(K):
        mn = jnp.min(m, axis=1, keepdims=True)
        sel = jnp.min(jnp.where(m == mn, iota, POOL), axis=1,
                      keepdims=True)
        simacc = jnp.where(col == j, mn, simacc)
        idxacc = jnp.where(col == j, sel, idxacc)
        m = jnp.where(iota == sel, jnp.inf, m)
    sim_ref[...] = simacc
    idx_ref[...] = idxacc


def _gather_body(prompts_hbm, idx_hbm, out_hbm, idx_v, bufs, gsems, ssems):
    wid = lax.axis_index("s") * NC + lax.axis_index("c")
    base = wid * ROWS_PER_W
    pltpu.sync_copy(idx_hbm.at[pl.ds(wid * NCH * IDX_STRIDE, NCH * IDX_STRIDE)],
                    idx_v)

    def gather(c):
        return pltpu.async_copy(
            prompts_hbm.at[idx_v.at[pl.ds(c * IDX_STRIDE, CHUNK)]],
            bufs[c % NBUF], gsems[c % NBUF])

    def scatter(c):
        return pltpu.async_copy(
            bufs[c % NBUF], out_hbm.at[pl.ds(base + c * CHUNK, CHUNK)],
            ssems[c % NBUF])

    g = [gather(b) for b in range(NBUF)]
    s = [None] * NCH
    for c in range(NCH):
        g[c % NBUF].wait()
        s[c] = scatter(c)
        if c + NBUF < NCH:
            s[c].wait()
            g[c % NBUF] = gather(c + NBUF)
    for c in range(NCH - NBUF, NCH):
        s[c].wait()


@jax.jit
def kernel(query, key_param, prompts):
    eps = 1e-8
    qn = jnp.maximum(jnp.linalg.norm(query, axis=-1, keepdims=True), eps)
    kn = jnp.maximum(jnp.linalg.norm(key_param, axis=-1, keepdims=True), eps)
    sim, idx = pl.pallas_call(
        _topk_body,
        out_shape=(
            jax.ShapeDtypeStruct((BATCH, K), jnp.float32),
            jax.ShapeDtypeStruct((BATCH, K), jnp.int32),
        ),
    )(query, key_param, qn, kn.T)

    gather = pl.kernel(
        _gather_body,
        out_type=jax.ShapeDtypeStruct((BATCH * K, PLEN, DIM), jnp.float32),
        mesh=plsc.VectorSubcoreMesh(core_axis_name="c", subcore_axis_name="s"),
        scratch_types=[
            pltpu.VMEM((NCH * IDX_STRIDE,), jnp.int32),
            [pltpu.VMEM((CHUNK, PLEN, DIM), jnp.float32) for _ in range(NBUF)],
            [pltpu.SemaphoreType.DMA for _ in range(NBUF)],
            [pltpu.SemaphoreType.DMA for _ in range(NBUF)],
        ],
    )
    idx_pad = jnp.pad(idx.reshape(BATCH * K // CHUNK, CHUNK),
                      ((0, 0), (0, IDX_STRIDE - CHUNK))).reshape(-1)
    sel_flat = gather(prompts, idx_pad)
    return sim, sel_flat.reshape(BATCH, K, PLEN, DIM)
